# baseline (speedup 1.0000x reference)
import jax
import jax.numpy as jnp
from jax import lax
from jax.experimental import pallas as pl
from jax.experimental.pallas import tpu as pltpu
from jax.experimental.pallas import tpu_sc as plsc

MAX_POSITIONS = 8192
HIDDEN_SIZE = 1024

NUM_CORES = 2
NUM_SUBCORES = 16
NUM_WORKERS = NUM_CORES * NUM_SUBCORES
ROWS_PER_WORKER = MAX_POSITIONS // NUM_WORKERS

CHUNK_ROWS = 8
NUM_CHUNKS = ROWS_PER_WORKER // CHUNK_ROWS
NUM_BUFS = 14
OUT_LAG = 6


def _copy_body(table_hbm, out_hbm, *scratch):
    bufs = scratch[:NUM_BUFS]
    sin = scratch[NUM_BUFS : 2 * NUM_BUFS]
    sout = scratch[2 * NUM_BUFS :]
    wid = lax.axis_index("s") * NUM_CORES + lax.axis_index("c")
    base = wid * ROWS_PER_WORKER

    in_cp = [None] * NUM_CHUNKS
    out_cp = [None] * NUM_CHUNKS
    for i in range(min(NUM_BUFS, NUM_CHUNKS)):
        in_cp[i] = pltpu.async_copy(
            table_hbm.at[pl.ds(base + i * CHUNK_ROWS, CHUNK_ROWS)],
            bufs[i],
            sin[i],
        )
    for i in range(NUM_CHUNKS):
        b = i % NUM_BUFS
        in_cp[i].wait()
        out_cp[i] = pltpu.async_copy(
            bufs[b],
            out_hbm.at[pl.ds(base + i * CHUNK_ROWS, CHUNK_ROWS)],
            sout[b],
        )
        k = i - OUT_LAG
        j = k + NUM_BUFS
        if k >= 0 and j < NUM_CHUNKS:
            out_cp[k].wait()
            in_cp[j] = pltpu.async_copy(
                table_hbm.at[pl.ds(base + j * CHUNK_ROWS, CHUNK_ROWS)],
                bufs[k % NUM_BUFS],
                sin[k % NUM_BUFS],
            )
    for i in range(max(0, NUM_CHUNKS - NUM_BUFS), NUM_CHUNKS):
        out_cp[i].wait()


@jax.jit
def _sc_copy(table):
    mesh = plsc.VectorSubcoreMesh(core_axis_name="c", subcore_axis_name="s")
    return pl.kernel(
        _copy_body,
        mesh=mesh,
        out_type=jax.ShapeDtypeStruct((MAX_POSITIONS, HIDDEN_SIZE), jnp.float32),
        scratch_types=(
            [pltpu.VMEM((CHUNK_ROWS, HIDDEN_SIZE), jnp.float32)] * NUM_BUFS
            + [pltpu.SemaphoreType.DMA] * (2 * NUM_BUFS)
        ),
    )(table)


def kernel(inputs, table):
    del inputs
    return _sc_copy(table)


# speedup vs baseline: 1.0012x; 1.0012x over previous
import jax
import jax.numpy as jnp
from jax import lax
from jax.experimental import pallas as pl
from jax.experimental.pallas import tpu as pltpu
from jax.experimental.pallas import tpu_sc as plsc

MAX_POSITIONS = 8192
HIDDEN_SIZE = 1024

NUM_CORES = 2
NUM_SUBCORES = 16
NUM_WORKERS = NUM_CORES * NUM_SUBCORES
ROWS_PER_WORKER = MAX_POSITIONS // NUM_WORKERS

CHUNK_ROWS = 16
NUM_CHUNKS = ROWS_PER_WORKER // CHUNK_ROWS
NUM_BUFS = 7
OUT_LAG = 5


def _copy_body(table_hbm, out_hbm, *scratch):
    bufs = scratch[:NUM_BUFS]
    sin = scratch[NUM_BUFS : 2 * NUM_BUFS]
    sout = scratch[2 * NUM_BUFS :]
    wid = lax.axis_index("s") * NUM_CORES + lax.axis_index("c")
    base = wid * ROWS_PER_WORKER

    in_cp = [None] * NUM_CHUNKS
    out_cp = [None] * NUM_CHUNKS
    for i in range(min(NUM_BUFS, NUM_CHUNKS)):
        in_cp[i] = pltpu.async_copy(
            table_hbm.at[pl.ds(base + i * CHUNK_ROWS, CHUNK_ROWS)],
            bufs[i],
            sin[i],
        )
    for i in range(NUM_CHUNKS):
        b = i % NUM_BUFS
        in_cp[i].wait()
        out_cp[i] = pltpu.async_copy(
            bufs[b],
            out_hbm.at[pl.ds(base + i * CHUNK_ROWS, CHUNK_ROWS)],
            sout[b],
        )
        k = i - OUT_LAG
        j = k + NUM_BUFS
        if k >= 0 and j < NUM_CHUNKS:
            out_cp[k].wait()
            in_cp[j] = pltpu.async_copy(
                table_hbm.at[pl.ds(base + j * CHUNK_ROWS, CHUNK_ROWS)],
                bufs[k % NUM_BUFS],
                sin[k % NUM_BUFS],
            )
    for i in range(max(0, NUM_CHUNKS - NUM_BUFS), NUM_CHUNKS):
        out_cp[i].wait()


@jax.jit
def _sc_copy(table):
    mesh = plsc.VectorSubcoreMesh(core_axis_name="c", subcore_axis_name="s")
    return pl.kernel(
        _copy_body,
        mesh=mesh,
        out_type=jax.ShapeDtypeStruct((MAX_POSITIONS, HIDDEN_SIZE), jnp.float32),
        scratch_types=(
            [pltpu.VMEM((CHUNK_ROWS, HIDDEN_SIZE), jnp.float32)] * NUM_BUFS
            + [pltpu.SemaphoreType.DMA] * (2 * NUM_BUFS)
        ),
    )(table)


def kernel(inputs, table):
    del inputs
    return _sc_copy(table)


# speedup vs baseline: 1.0268x; 1.0255x over previous
import jax
import jax.numpy as jnp
from jax import lax
from jax.experimental import pallas as pl
from jax.experimental.pallas import tpu as pltpu
from jax.experimental.pallas import tpu_sc as plsc

MAX_POSITIONS = 8192
HIDDEN_SIZE = 1024

NUM_CORES = 2
NUM_SUBCORES = 16
NUM_WORKERS = NUM_CORES * NUM_SUBCORES
ROWS_PER_WORKER = MAX_POSITIONS // NUM_WORKERS

CHUNK_ROWS = 16
NUM_CHUNKS = ROWS_PER_WORKER // CHUNK_ROWS
NUM_BUFS = 7
OUT_LAG = 2


def _copy_body(table_hbm, out_hbm, *scratch):
    bufs = scratch[:NUM_BUFS]
    sin = scratch[NUM_BUFS : 2 * NUM_BUFS]
    sout = scratch[2 * NUM_BUFS :]
    wid = lax.axis_index("s") * NUM_CORES + lax.axis_index("c")
    base = wid * ROWS_PER_WORKER

    in_cp = [None] * NUM_CHUNKS
    out_cp = [None] * NUM_CHUNKS
    for i in range(min(NUM_BUFS, NUM_CHUNKS)):
        in_cp[i] = pltpu.async_copy(
            table_hbm.at[pl.ds(base + i * CHUNK_ROWS, CHUNK_ROWS)],
            bufs[i],
            sin[i],
        )
    for i in range(NUM_CHUNKS):
        b = i % NUM_BUFS
        in_cp[i].wait()
        out_cp[i] = pltpu.async_copy(
            bufs[b],
            out_hbm.at[pl.ds(base + i * CHUNK_ROWS, CHUNK_ROWS)],
            sout[b],
        )
        k = i - OUT_LAG
        j = k + NUM_BUFS
        if k >= 0 and j < NUM_CHUNKS:
            out_cp[k].wait()
            in_cp[j] = pltpu.async_copy(
                table_hbm.at[pl.ds(base + j * CHUNK_ROWS, CHUNK_ROWS)],
                bufs[k % NUM_BUFS],
                sin[k % NUM_BUFS],
            )
    for i in range(max(0, NUM_CHUNKS - NUM_BUFS), NUM_CHUNKS):
        out_cp[i].wait()


@jax.jit
def _sc_copy(table):
    mesh = plsc.VectorSubcoreMesh(core_axis_name="c", subcore_axis_name="s")
    return pl.kernel(
        _copy_body,
        mesh=mesh,
        out_type=jax.ShapeDtypeStruct((MAX_POSITIONS, HIDDEN_SIZE), jnp.float32),
        scratch_types=(
            [pltpu.VMEM((CHUNK_ROWS, HIDDEN_SIZE), jnp.float32)] * NUM_BUFS
            + [pltpu.SemaphoreType.DMA] * (2 * NUM_BUFS)
        ),
    )(table)


def kernel(inputs, table):
    del inputs
    return _sc_copy(table)
